# Initial kernel scaffold; baseline (speedup 1.0000x reference)
import functools

import jax
import jax.numpy as jnp
from jax import lax
from jax.experimental import pallas as pl
from jax.experimental.pallas import tpu as pltpu
from jax.experimental.pallas import tpu_sc as plsc

N = 10000
D = 128
E = 320000

NC = 2
NS = 16
NW = NC * NS
EDGES_PER_TILE = E // NW
CHUNK = 80
NCHUNKS = EDGES_PER_TILE // CHUNK
ROWS_PER_TILE = N // NS
NPAD = NS * 640

_MESH = plsc.VectorSubcoreMesh(core_axis_name="c", subcore_axis_name="s")


def _sc_pass(table, src, dst, zeros_nd, zeros_n, ones_c, with_deg):
  out_type = [jax.ShapeDtypeStruct((NC, N, D), jnp.float32)]
  scratch = [
      pltpu.VMEM_SHARED((N, D), jnp.float32),
      pltpu.VMEM((CHUNK,), jnp.int32),
      pltpu.VMEM((CHUNK,), jnp.int32),
      pltpu.VMEM((CHUNK, D), jnp.float32),
      pltpu.SemaphoreType.DMA,
  ]
  if with_deg:
    out_type.append(jax.ShapeDtypeStruct((NC, NPAD), jnp.float32))
    scratch.append(pltpu.VMEM_SHARED((NPAD,), jnp.float32))
    scratch.append(pltpu.VMEM((CHUNK,), jnp.float32))

  def body(table_hbm, src_hbm, dst_hbm, znd_hbm, zn_hbm, ones_hbm,
           *outs_and_scratch):
    if with_deg:
      out_h, out_deg, acc, idx_v, dst_v, rows_v, sem, deg, ones_v = (
          outs_and_scratch)
    else:
      out_h, acc, idx_v, dst_v, rows_v, sem = outs_and_scratch
    c = lax.axis_index("c")
    s = lax.axis_index("s")
    wid = c * NS + s

    pltpu.sync_copy(znd_hbm.at[pl.ds(s * ROWS_PER_TILE, ROWS_PER_TILE)],
                    acc.at[pl.ds(s * ROWS_PER_TILE, ROWS_PER_TILE)])
    if with_deg:
      pltpu.sync_copy(zn_hbm.at[pl.ds(s * 640, 640)],
                      deg.at[pl.ds(s * 640, 640)])
      pltpu.sync_copy(ones_hbm, ones_v)
    plsc.subcore_barrier()

    base = wid * EDGES_PER_TILE

    def step(k, carry):
      off = base + k * CHUNK
      pltpu.sync_copy(src_hbm.at[pl.ds(off, CHUNK)], idx_v)
      pltpu.sync_copy(dst_hbm.at[pl.ds(off, CHUNK)], dst_v)
      pltpu.async_copy(table_hbm.at[idx_v], rows_v, sem).wait()
      pltpu.sync_copy(rows_v, acc.at[dst_v], add=True)
      if with_deg:
        pltpu.sync_copy(ones_v, deg.at[dst_v], add=True)
      return carry

    lax.fori_loop(0, NCHUNKS, step, 0)
    plsc.subcore_barrier()

    pltpu.sync_copy(acc.at[pl.ds(s * ROWS_PER_TILE, ROWS_PER_TILE)],
                    out_h.at[c, pl.ds(s * ROWS_PER_TILE, ROWS_PER_TILE)])
    if with_deg:
      pltpu.sync_copy(deg.at[pl.ds(s * 640, 640)],
                      out_deg.at[c, pl.ds(s * 640, 640)])

  fn = pl.kernel(body, out_type=out_type, mesh=_MESH, scratch_types=scratch)
  return fn(table, src, dst, zeros_nd, zeros_n, ones_c)


def _combine_body(pa_ref, pd_ref, out_ref):
  total = pa_ref[0] + pa_ref[1]
  deg = pd_ref[0] + pd_ref[1]
  inv = 1.0 / jnp.maximum(deg, 1.0)
  out_ref[...] = total * inv


_ROWB = 1000


def _combine(pa, pd3):
  grid = (N // _ROWB,)
  return pl.pallas_call(
      _combine_body,
      grid=grid,
      in_specs=[
          pl.BlockSpec((NC, _ROWB, D), lambda i: (0, i, 0)),
          pl.BlockSpec((NC, _ROWB, 1), lambda i: (0, i, 0)),
      ],
      out_specs=pl.BlockSpec((_ROWB, D), lambda i: (i, 0)),
      out_shape=jax.ShapeDtypeStruct((N, D), jnp.float32),
  )(pa, pd3)


def kernel(x, edge_index):
  ei = edge_index.astype(jnp.int32)
  src = ei[0]
  dst = ei[1]
  zeros_nd = jnp.zeros((N, D), jnp.float32)
  zeros_n = jnp.zeros((NPAD,), jnp.float32)
  ones_c = jnp.ones((CHUNK,), jnp.float32)

  ph, pdeg = _sc_pass(x, src, dst, zeros_nd, zeros_n, ones_c, with_deg=True)
  pd3 = pdeg[:, :N, None]
  h = _combine(ph, pd3)
  (ph2,) = _sc_pass(h, src, dst, zeros_nd, zeros_n, ones_c, with_deg=False)
  return _combine(ph2, pd3)


# speedup vs baseline: 5.4972x; 5.4972x over previous
import functools

import jax
import jax.numpy as jnp
from jax import lax
from jax.experimental import pallas as pl
from jax.experimental.pallas import tpu as pltpu
from jax.experimental.pallas import tpu_sc as plsc

N = 10000
D = 128
E = 320000

NC = 2
NS = 16
NW = NC * NS
EDGES_PER_TILE = E // NW
CHUNK = 80
NCHUNKS = EDGES_PER_TILE // CHUNK
NPAD = NS * 640
ROWS_PER_TILE = NPAD // NS

_MESH = plsc.VectorSubcoreMesh(core_axis_name="c", subcore_axis_name="s")


def _sc_pass(table, src, dst, zeros_nd, zeros_n, ones_c, with_deg):
  out_type = [jax.ShapeDtypeStruct((NC, NPAD, D), jnp.float32)]
  scratch = [
      pltpu.VMEM_SHARED((NPAD, D), jnp.float32),
      pltpu.VMEM((CHUNK,), jnp.int32),
      pltpu.VMEM((CHUNK,), jnp.int32),
      pltpu.VMEM((CHUNK, D), jnp.float32),
      pltpu.SemaphoreType.DMA,
  ]
  if with_deg:
    out_type.append(jax.ShapeDtypeStruct((NC, NPAD), jnp.float32))
    scratch.append(pltpu.VMEM_SHARED((NPAD,), jnp.float32))
    scratch.append(pltpu.VMEM((CHUNK,), jnp.float32))

  def body(table_hbm, src_hbm, dst_hbm, znd_hbm, zn_hbm, ones_hbm,
           *outs_and_scratch):
    if with_deg:
      out_h, out_deg, acc, idx_v, dst_v, rows_v, sem, deg, ones_v = (
          outs_and_scratch)
    else:
      out_h, acc, idx_v, dst_v, rows_v, sem = outs_and_scratch
    c = lax.axis_index("c")
    s = lax.axis_index("s")
    wid = c * NS + s

    pltpu.sync_copy(znd_hbm.at[pl.ds(s * ROWS_PER_TILE, ROWS_PER_TILE)],
                    acc.at[pl.ds(s * ROWS_PER_TILE, ROWS_PER_TILE)])
    if with_deg:
      pltpu.sync_copy(zn_hbm.at[pl.ds(s * 640, 640)],
                      deg.at[pl.ds(s * 640, 640)])
      pltpu.sync_copy(ones_hbm, ones_v)
    plsc.subcore_barrier()

    base = wid * EDGES_PER_TILE

    def step(k, carry):
      off = base + k * CHUNK
      pltpu.sync_copy(src_hbm.at[pl.ds(off, CHUNK)], idx_v)
      pltpu.sync_copy(dst_hbm.at[pl.ds(off, CHUNK)], dst_v)
      pltpu.async_copy(table_hbm.at[idx_v], rows_v, sem).wait()
      pltpu.sync_copy(rows_v, acc.at[dst_v], add=True)
      if with_deg:
        pltpu.sync_copy(ones_v, deg.at[dst_v], add=True)
      return carry

    lax.fori_loop(0, NCHUNKS, step, 0)
    plsc.subcore_barrier()

    pltpu.sync_copy(acc.at[pl.ds(s * ROWS_PER_TILE, ROWS_PER_TILE)],
                    out_h.at[c, pl.ds(s * ROWS_PER_TILE, ROWS_PER_TILE)])
    if with_deg:
      pltpu.sync_copy(deg.at[pl.ds(s * 640, 640)],
                      out_deg.at[c, pl.ds(s * 640, 640)])

  fn = pl.kernel(body, out_type=out_type, mesh=_MESH, scratch_types=scratch)
  return fn(table, src, dst, zeros_nd, zeros_n, ones_c)


def _combine_body(pa_ref, pd_ref, out_ref):
  total = pa_ref[0] + pa_ref[1]
  deg = pd_ref[0] + pd_ref[1]
  inv = 1.0 / jnp.maximum(deg, 1.0)
  out_ref[...] = total * inv


_ROWB = 1024


def _combine(pa, pd3):
  grid = (NPAD // _ROWB,)
  return pl.pallas_call(
      _combine_body,
      grid=grid,
      in_specs=[
          pl.BlockSpec((NC, _ROWB, D), lambda i: (0, i, 0)),
          pl.BlockSpec((NC, _ROWB, 1), lambda i: (0, i, 0)),
      ],
      out_specs=pl.BlockSpec((_ROWB, D), lambda i: (i, 0)),
      out_shape=jax.ShapeDtypeStruct((NPAD, D), jnp.float32),
  )(pa, pd3)


def kernel(x, edge_index):
  ei = edge_index.astype(jnp.int32)
  src = ei[0]
  dst = ei[1]
  zeros_nd = jnp.zeros((NPAD, D), jnp.float32)
  zeros_n = jnp.zeros((NPAD,), jnp.float32)
  ones_c = jnp.ones((CHUNK,), jnp.float32)

  ph, pdeg = _sc_pass(x, src, dst, zeros_nd, zeros_n, ones_c, with_deg=True)
  pd3 = pdeg[:, :, None]
  h = _combine(ph, pd3)
  (ph2,) = _sc_pass(h, src, dst, zeros_nd, zeros_n, ones_c, with_deg=False)
  return _combine(ph2, pd3)[:N]
